# Initial kernel scaffold; baseline (speedup 1.0000x reference)
import jax
import jax.numpy as jnp
from jax.experimental import pallas as pl


def kernel(bp_scores, sequences, sequence_lengths):
    raise NotImplementedError("write your pallas kernel here")


# speedup vs baseline: 2248.4800x; 2248.4800x over previous
import functools

import numpy as np
import jax
import jax.numpy as jnp
from jax.experimental import pallas as pl
from jax.experimental.pallas import tpu as pltpu

MIN_BP = 3


def _build_table16():
    V = np.zeros((4, 4), dtype=np.float32)
    for (a, b) in [(0, 3), (3, 0), (3, 2), (2, 1), (2, 3), (1, 2)]:
        V[a, b] = 1.0
    SE = 0.5 * V[:, :, None, None] * V[None, None, :, :]
    stacking = {
        (0, 3, 0, 3): 0.9, (0, 3, 2, 1): 1.1, (0, 3, 2, 3): 0.8,
        (2, 1, 0, 3): 1.1, (2, 1, 2, 1): 1.3, (2, 1, 2, 3): 1.0,
        (2, 3, 0, 3): 0.8, (2, 3, 2, 1): 1.0, (2, 3, 2, 3): 0.7,
    }
    for k, v in stacking.items():
        SE[k] = v
    return np.ascontiguousarray(SE.transpose(0, 2, 1, 3).reshape(16, 16))


_T16 = _build_table16()


def _body(R, L, lens_ref, bp_ref, idxf_ref, idxr_ref, tab_ref, out_ref):
    f32 = jnp.float32
    b = pl.program_id(0)
    ib = pl.program_id(1)
    ln = lens_ref[b]

    irow = ib * R + jax.lax.broadcasted_iota(jnp.int32, (R, 1), 0)
    jcol = jax.lax.broadcasted_iota(jnp.int32, (1, L), 1)
    d = jcol - irow

    k_row = jax.lax.broadcasted_iota(jnp.int32, (1, 16), 1)
    k_col = jax.lax.broadcasted_iota(jnp.int32, (16, 1), 0)

    T = tab_ref[0]
    Tt = tab_ref[1]

    def rvec(r):
        return idxr_ref[0, r, :].reshape(R, 1)

    def cvec(r):
        return idxf_ref[0, r, :].reshape(1, L)

    def row_oh(vals, factor):
        return ((vals == k_row) & factor).astype(f32)

    def col_oh(vals, factor):
        return ((k_col == vals) & factor).astype(f32)

    def term(tab, r_vals, r_fac, c_vals, c_fac):
        P = jnp.dot(row_oh(r_vals, r_fac), tab, preferred_element_type=f32)
        return jnp.dot(P, col_oh(c_vals, c_fac), preferred_element_type=f32)

    u_r, v_r, um1_r, vp1_r = rvec(0), rvec(1), rvec(2), rvec(3)
    u_c, v_c, um1_c, vp1_c = cvec(0), cvec(1), cvec(2), cvec(3)

    t1 = term(T, u_r, irow < ln - 1, v_c, jcol < ln)
    t2 = term(Tt, v_r, irow < ln, u_c, jcol < ln - 1)
    t3 = term(T, um1_r, (irow < ln) & (irow >= 1),
              vp1_c, (jcol < ln - 1) & (jcol <= L - 2))
    t4 = term(Tt, vp1_r, (irow < ln - 1) & (irow <= L - 2),
              um1_c, (jcol < ln) & (jcol >= 1))

    acc = (t1 * (d > MIN_BP).astype(f32) + t2 * (d < -MIN_BP).astype(f32)
           + t3 * (d > MIN_BP - 2).astype(f32)
           + t4 * (d < -(MIN_BP - 2)).astype(f32))
    out_ref[0] = bp_ref[0] + acc


@jax.jit
def kernel(bp_scores, sequences, sequence_lengths):
    B, L, _ = bp_scores.shape
    R = min(256, L)
    NI = L // R

    s = sequences.astype(jnp.int32)
    ni = jnp.roll(s, -1, axis=1)
    pj = jnp.roll(s, 1, axis=1)
    u = 4 * s + ni
    v = 4 * s + pj
    um1 = jnp.roll(u, 1, axis=1)
    vp1 = jnp.roll(v, -1, axis=1)
    idx = jnp.stack([u, v, um1, vp1, u, v, um1, vp1], axis=1)

    T = jnp.asarray(_T16)
    tab = jnp.stack([T, T.T])
    lens = sequence_lengths.astype(jnp.int32)

    grid_spec = pltpu.PrefetchScalarGridSpec(
        num_scalar_prefetch=1,
        grid=(B, NI),
        in_specs=[
            pl.BlockSpec((1, R, L), lambda b, i, *_: (b, i, 0)),
            pl.BlockSpec((1, 8, L), lambda b, i, *_: (b, 0, 0)),
            pl.BlockSpec((1, 8, R), lambda b, i, *_: (b, 0, i)),
            pl.BlockSpec((2, 16, 16), lambda b, i, *_: (0, 0, 0)),
        ],
        out_specs=pl.BlockSpec((1, R, L), lambda b, i, *_: (b, i, 0)),
    )
    return pl.pallas_call(
        functools.partial(_body, R, L),
        grid_spec=grid_spec,
        out_shape=jax.ShapeDtypeStruct((B, L, L), bp_scores.dtype),
        compiler_params=pltpu.CompilerParams(
            dimension_semantics=("parallel", "parallel")),
    )(lens, bp_scores, idx, idx, tab)
